# baseline (speedup 1.0000x reference)
from functools import partial

import numpy as np

import jax
import jax.numpy as jnp
from jax.experimental import pallas as pl
from jax.experimental.pallas import tpu as pltpu


def _fused_kernel(x_ref, x2_ref, w_ref, s_ref, b_ref, o_ref, *, cout, tg, lw):
    c2r = 2 * cout
    cin = x_ref.shape[0]
    x2d = x_ref[...].reshape(cin, tg * lw)
    acc = jnp.dot(w_ref[...], x2d,
                  preferred_element_type=jnp.float32)
    b = b_ref[...]
    for g in range(tg):
        a0 = acc[:c2r, g * lw:(g + 1) * lw]
        a1 = acc[c2r:, g * lw:(g + 1) * lw]
        a = jnp.concatenate([a0, a1], axis=1)
        sg = jnp.dot(a, s_ref[...],
                     preferred_element_type=jnp.float32)
        top = sg[:cout]
        bot = sg[cout:]
        chunk = jnp.concatenate(
            [top[:, :lw], bot[:, :lw], top[:, lw:], bot[:, lw:]], axis=1) + b
        o_ref[:cout, g * 4 * lw:(g + 1) * 4 * lw] = chunk.astype(o_ref.dtype)
    o_ref[cout:, :] = x2_ref[...].astype(o_ref.dtype)


def kernel(x1, x2, weight, bias):
    N, Cin, H, W = x1.shape
    Cout = weight.shape[1]
    _, C2, H2, W2 = x2.shape
    assert (H2, W2) == (2 * H, 2 * W) and x2.shape[0] == N
    Ctot = Cout + C2
    dt = x1.dtype
    lw = 2 * W

    half_h = H // 2
    tg = 1
    for cand in (16, 8, 4, 2, 1):
        if half_h % cand == 0:
            tg = cand
            break
    if half_h >= 32 and half_h % 32 == 0:
        tg = 32

    w4 = jnp.transpose(weight, (3, 2, 1, 0)).reshape(4 * Cout, Cin).astype(dt)
    b2 = bias.astype(jnp.float32).reshape(Cout, 1)

    s_np = np.zeros((2 * lw, 2 * lw), np.float32)
    jj, pp, ww = np.meshgrid(np.arange(2), np.arange(2), np.arange(W),
                             indexing="ij")
    s_np[(jj * lw + pp * W + ww).ravel(), (pp * lw + 2 * ww + jj).ravel()] = 1.0
    sj = jnp.asarray(s_np)

    x2f = x2.reshape(N, C2, 4 * H * W)

    out = pl.pallas_call(
        partial(_fused_kernel, cout=Cout, tg=tg, lw=lw),
        out_shape=jax.ShapeDtypeStruct((N, Ctot, 4 * H * W), dt),
        grid_spec=pltpu.PrefetchScalarGridSpec(
            num_scalar_prefetch=0,
            grid=(N, half_h // tg),
            in_specs=[
                pl.BlockSpec((None, Cin, 2 * tg, W), lambda n, t: (n, 0, t, 0)),
                pl.BlockSpec((None, C2, tg * 4 * lw), lambda n, t: (n, 0, t)),
                pl.BlockSpec((4 * Cout, Cin), lambda n, t: (0, 0)),
                pl.BlockSpec((2 * lw, 2 * lw), lambda n, t: (0, 0)),
                pl.BlockSpec((Cout, 1), lambda n, t: (0, 0)),
            ],
            out_specs=pl.BlockSpec((None, Ctot, tg * 4 * lw),
                                   lambda n, t: (n, 0, t)),
        ),
        compiler_params=pltpu.CompilerParams(
            dimension_semantics=("parallel", "parallel"),
        ),
    )(x1, x2f, w4, sj, b2)

    return out.reshape(N, Ctot, 2 * H, 2 * W)


# speedup vs baseline: 2.9970x; 2.9970x over previous
from functools import partial

import numpy as np

import jax
import jax.numpy as jnp
from jax.experimental import pallas as pl
from jax.experimental.pallas import tpu as pltpu


def _fused_kernel(x_ref, x2_ref, w_ref, s_ref, b_ref, o_ref, *, cout, tg, lw):
    c2r = 2 * cout
    acc = jnp.dot(w_ref[...], x_ref[...],
                  preferred_element_type=jnp.float32)
    b = b_ref[...]
    for m in range(tg // 2):
        pieces = []
        for g in (2 * m, 2 * m + 1):
            a0 = acc[:c2r, g * lw:(g + 1) * lw]
            a1 = acc[c2r:, g * lw:(g + 1) * lw]
            a = jnp.concatenate([a0, a1], axis=1)
            sg = jnp.dot(a, s_ref[...],
                         preferred_element_type=jnp.float32)
            top = sg[:cout]
            bot = sg[cout:]
            pieces += [top[:, :lw], bot[:, :lw], top[:, lw:], bot[:, lw:]]
        stack = jnp.stack(pieces, axis=0)
        chunk = jnp.transpose(stack, (1, 0, 2)) + b
        o_ref[:cout, 8 * m:8 * m + 8, :] = chunk.astype(o_ref.dtype)
    o_ref[cout:, :, :] = x2_ref[...].astype(o_ref.dtype)


def kernel(x1, x2, weight, bias):
    N, Cin, H, W = x1.shape
    Cout = weight.shape[1]
    _, C2, H2, W2 = x2.shape
    assert (H2, W2) == (2 * H, 2 * W) and x2.shape[0] == N
    Ctot = Cout + C2
    dt = x1.dtype
    lw = 2 * W

    half_h = H // 2
    tg = 2
    for cand in (32, 16, 8, 4, 2):
        if half_h % cand == 0:
            tg = cand
            break

    w4 = jnp.transpose(weight, (3, 2, 1, 0)).reshape(4 * Cout, Cin).astype(dt)
    b3 = bias.astype(jnp.float32).reshape(Cout, 1, 1)

    s_np = np.zeros((2 * lw, 2 * lw), np.float32)
    jj, pp, ww = np.meshgrid(np.arange(2), np.arange(2), np.arange(W),
                             indexing="ij")
    s_np[(jj * lw + pp * W + ww).ravel(), (pp * lw + 2 * ww + jj).ravel()] = 1.0
    sj = jnp.asarray(s_np)

    xf = x1.reshape(N, Cin, H * W)

    out = pl.pallas_call(
        partial(_fused_kernel, cout=Cout, tg=tg, lw=lw),
        out_shape=jax.ShapeDtypeStruct((N, Ctot, 2 * H, 2 * W), dt),
        grid_spec=pltpu.PrefetchScalarGridSpec(
            num_scalar_prefetch=0,
            grid=(N, half_h // tg),
            in_specs=[
                pl.BlockSpec((None, Cin, tg * lw), lambda n, t: (n, 0, t)),
                pl.BlockSpec((None, C2, 4 * tg, lw), lambda n, t: (n, 0, t, 0)),
                pl.BlockSpec((4 * Cout, Cin), lambda n, t: (0, 0)),
                pl.BlockSpec((2 * lw, 2 * lw), lambda n, t: (0, 0)),
                pl.BlockSpec((Cout, 1, 1), lambda n, t: (0, 0, 0)),
            ],
            out_specs=pl.BlockSpec((None, Ctot, 4 * tg, lw),
                                   lambda n, t: (n, 0, t, 0)),
        ),
        compiler_params=pltpu.CompilerParams(
            dimension_semantics=("parallel", "parallel"),
        ),
    )(xf, x2, w4, sj, b3)

    return out
